# baseline (speedup 1.0000x reference)
import functools

import jax
import jax.numpy as jnp
from jax import lax
from jax.experimental import pallas as pl
from jax.experimental.pallas import tpu as pltpu
from jax.experimental.pallas import tpu_sc as plsc

T, D, F, E = 4096, 2048, 2048, 8
EP = 128
TB = 512
BQ = 256
NB = T // BQ + E
NBQ = NB * BQ
TF = 1024
NF = F // TF
TD = 1024
ND = D // TD
BE_NONE = 127
NTB = T // BQ

_f32 = jnp.float32
_bf16 = jnp.bfloat16
_i32 = jnp.int32


def _router_body(x_ref, wr_ref, eid_ref, pos_ref, score_ref, xs_ref,
                 counts_ref, csum_ref):
    b = pl.program_id(0)
    x = x_ref[...]
    logits = jnp.dot(x, wr_ref[...], preferred_element_type=_f32)
    col = lax.broadcasted_iota(_i32, (TB, EP), 1)
    logits = jnp.where(col < E, logits, -jnp.inf)
    m = jnp.max(logits, axis=1, keepdims=True)
    eid = jnp.min(jnp.where(logits == m, col, EP), axis=1, keepdims=True)
    score = 1.0 / (1.0 + jnp.exp(-m))
    onehot = (col == eid).astype(_f32)
    rio = lax.broadcasted_iota(_i32, (TB, TB), 0)
    cio = lax.broadcasted_iota(_i32, (TB, TB), 1)
    tri = (cio < rio).astype(_f32)
    posmat = jnp.dot(tri, onehot, preferred_element_type=_f32)

    @pl.when(b == 0)
    def _():
        csum_ref[...] = jnp.zeros_like(csum_ref)

    prev = csum_ref[...]
    pos = jnp.sum((posmat + prev) * onehot, axis=1, keepdims=True)
    csum_ref[...] = prev + jnp.sum(onehot, axis=0, keepdims=True)
    eid_ref[...] = eid
    pos_ref[...] = pos.astype(_i32)
    score_ref[...] = score
    xs_ref[...] = x * score
    counts_ref[...] = csum_ref[...]


def _router(x, wr_pad):
    return pl.pallas_call(
        _router_body,
        grid=(T // TB,),
        in_specs=[
            pl.BlockSpec((TB, D), lambda b: (b, 0)),
            pl.BlockSpec((D, EP), lambda b: (0, 0)),
        ],
        out_specs=[
            pl.BlockSpec((TB, 1), lambda b: (b, 0)),
            pl.BlockSpec((TB, 1), lambda b: (b, 0)),
            pl.BlockSpec((TB, 1), lambda b: (b, 0)),
            pl.BlockSpec((TB, D), lambda b: (b, 0)),
            pl.BlockSpec((1, EP), lambda b: (0, 0)),
        ],
        out_shape=[
            jax.ShapeDtypeStruct((T, 1), _i32),
            jax.ShapeDtypeStruct((T, 1), _i32),
            jax.ShapeDtypeStruct((T, 1), _f32),
            jax.ShapeDtypeStruct((T, D), _f32),
            jax.ShapeDtypeStruct((1, EP), _f32),
        ],
        scratch_shapes=[pltpu.VMEM((1, EP), _f32)],
        compiler_params=pltpu.CompilerParams(
            dimension_semantics=("arbitrary",)),
    )(x, wr_pad)


def _book_body(cnt_ref, eid_ref, pos_ref, slot_ref, be_ref):
    eid = eid_ref[...]
    pos = pos_ref[...]
    base = jnp.zeros((T // TB, TB), _i32)
    beacc = jnp.zeros((1, EP), _i32)
    bio = lax.broadcasted_iota(_i32, (1, EP), 1)
    bs = _i32(0)
    for e in range(E):
        c = cnt_ref[0, e]
        nb = (c + BQ - 1) // BQ
        base = base + jnp.where(eid == e, bs * BQ, 0)
        bs = bs + nb
        beacc = beacc + (bio >= bs).astype(_i32)
    slot_ref[...] = base + pos
    be_ref[...] = jnp.where(beacc >= E, BE_NONE, beacc)


def _bookkeep(counts_i, eid2, pos2):
    return pl.pallas_call(
        _book_body,
        in_specs=[
            pl.BlockSpec(memory_space=pltpu.SMEM),
            pl.BlockSpec((T // TB, TB), lambda: (0, 0)),
            pl.BlockSpec((T // TB, TB), lambda: (0, 0)),
        ],
        out_specs=[
            pl.BlockSpec((T // TB, TB), lambda: (0, 0)),
            pl.BlockSpec((1, EP), lambda: (0, 0)),
        ],
        out_shape=[
            jax.ShapeDtypeStruct((T // TB, TB), _i32),
            jax.ShapeDtypeStruct((1, EP), _i32),
        ],
    )(counts_i, eid2, pos2)


_NC, _NS = 2, 16
_NW = _NC * _NS
_PW = T // _NW
_CS = 32


def _sc_scatter(xs, slot):
    mesh = plsc.VectorSubcoreMesh(core_axis_name="c", subcore_axis_name="s")

    @functools.partial(
        pl.kernel,
        out_type=jax.ShapeDtypeStruct((NBQ, D), _f32),
        mesh=mesh,
        scratch_types=[
            pltpu.VMEM((_CS,), _i32),
            pltpu.VMEM((_CS, D), _f32),
            pltpu.SemaphoreType.DMA,
        ],
    )
    def k(xs_hbm, slot_hbm, out_hbm, idx_v, rows_v, sem):
        wid = lax.axis_index("s") * _NC + lax.axis_index("c")
        base = wid * _PW

        def body(j, carry):
            off = base + j * _CS
            pltpu.sync_copy(slot_hbm.at[pl.ds(off, _CS)], idx_v)
            pltpu.sync_copy(xs_hbm.at[pl.ds(off, _CS)], rows_v)
            pltpu.async_copy(rows_v, out_hbm.at[idx_v], sem).wait()
            return carry

        lax.fori_loop(0, _PW // _CS, body, 0)

    return k(xs, slot)


def _sc_gather(out_sorted, slot):
    mesh = plsc.VectorSubcoreMesh(core_axis_name="c", subcore_axis_name="s")

    @functools.partial(
        pl.kernel,
        out_type=jax.ShapeDtypeStruct((T, D), _f32),
        mesh=mesh,
        scratch_types=[
            pltpu.VMEM((_CS,), _i32),
            pltpu.VMEM((_CS, D), _f32),
            pltpu.SemaphoreType.DMA,
        ],
    )
    def k(src_hbm, slot_hbm, out_hbm, idx_v, rows_v, sem):
        wid = lax.axis_index("s") * _NC + lax.axis_index("c")
        base = wid * _PW

        def body(j, carry):
            off = base + j * _CS
            pltpu.sync_copy(slot_hbm.at[pl.ds(off, _CS)], idx_v)
            pltpu.async_copy(src_hbm.at[idx_v], rows_v, sem).wait()
            pltpu.sync_copy(rows_v, out_hbm.at[pl.ds(off, _CS)])
            return carry

        lax.fori_loop(0, _PW // _CS, body, 0)

    return k(out_sorted, slot)


def _upr_body(be_ref, x_ref, wg_ref, wu_ref, h_ref, wg16, wu16):
    b = pl.program_id(1)
    e = be_ref[b]
    eprev = be_ref[jnp.maximum(b - 1, 0)]

    @pl.when(e != BE_NONE)
    def _():
        @pl.when((b == 0) | (e != eprev))
        def _():
            wg16[...] = wg_ref[0].astype(_bf16)
            wu16[...] = wu_ref[0].astype(_bf16)

        x16 = x_ref[...].astype(_bf16)
        g = jnp.dot(x16, wg16[...], preferred_element_type=_f32)
        u = jnp.dot(x16, wu16[...], preferred_element_type=_f32)
        h = g / (1.0 + jnp.exp(-g)) * u
        h_ref[...] = h.astype(_bf16)


def _up_routed(be, x_sorted, w_gate, w_up):
    grid_spec = pltpu.PrefetchScalarGridSpec(
        num_scalar_prefetch=1,
        grid=(NF, NB),
        in_specs=[
            pl.BlockSpec((BQ, D),
                         lambda fo, b, be: (jnp.where(be[b] == BE_NONE, 0, b),
                                            0)),
            pl.BlockSpec((1, D, TF),
                         lambda fo, b, be: (jnp.minimum(be[b], E - 1), 0, fo)),
            pl.BlockSpec((1, D, TF),
                         lambda fo, b, be: (jnp.minimum(be[b], E - 1), 0, fo)),
        ],
        out_specs=pl.BlockSpec((BQ, TF), lambda fo, b, be: (b, fo)),
        scratch_shapes=[pltpu.VMEM((D, TF), _bf16),
                        pltpu.VMEM((D, TF), _bf16)],
    )
    return pl.pallas_call(
        _upr_body,
        grid_spec=grid_spec,
        out_shape=jax.ShapeDtypeStruct((NBQ, F), _bf16),
        compiler_params=pltpu.CompilerParams(
            dimension_semantics=("arbitrary", "arbitrary")),
    )(be, x_sorted, w_gate, w_up)


def _downr_body(be_ref, h_ref, wd_ref, o_ref, wd16):
    b = pl.program_id(1)
    e = be_ref[b]
    eprev = be_ref[jnp.maximum(b - 1, 0)]

    @pl.when(e != BE_NONE)
    def _():
        @pl.when((b == 0) | (e != eprev))
        def _():
            wd16[...] = wd_ref[0].astype(_bf16)

        o_ref[...] = jnp.dot(h_ref[...], wd16[...],
                             preferred_element_type=_f32)


def _down_routed(be, h, w_down):
    grid_spec = pltpu.PrefetchScalarGridSpec(
        num_scalar_prefetch=1,
        grid=(ND, NB),
        in_specs=[
            pl.BlockSpec((BQ, F),
                         lambda dc, b, be: (jnp.where(be[b] == BE_NONE, 0, b),
                                            0)),
            pl.BlockSpec((1, F, TD),
                         lambda dc, b, be: (jnp.minimum(be[b], E - 1), 0, dc)),
        ],
        out_specs=pl.BlockSpec((BQ, TD), lambda dc, b, be: (b, dc)),
        scratch_shapes=[pltpu.VMEM((F, TD), _bf16)],
    )
    return pl.pallas_call(
        _downr_body,
        grid_spec=grid_spec,
        out_shape=jax.ShapeDtypeStruct((NBQ, D), _f32),
        compiler_params=pltpu.CompilerParams(
            dimension_semantics=("arbitrary", "arbitrary")),
    )(be, h, w_down)


def _ups_body(x_ref, wg_ref, wu_ref, h_ref, wg16, wu16):
    tb = pl.program_id(1)

    @pl.when(tb == 0)
    def _():
        wg16[...] = wg_ref[...].astype(_bf16)
        wu16[...] = wu_ref[...].astype(_bf16)

    x16 = x_ref[...].astype(_bf16)
    g = jnp.dot(x16, wg16[...], preferred_element_type=_f32)
    u = jnp.dot(x16, wu16[...], preferred_element_type=_f32)
    h = g / (1.0 + jnp.exp(-g)) * u
    h_ref[...] = h.astype(_bf16)


def _up_shared(x, ws_gate, ws_up):
    return pl.pallas_call(
        _ups_body,
        grid=(NF, NTB),
        in_specs=[
            pl.BlockSpec((BQ, D), lambda fo, tb: (tb, 0)),
            pl.BlockSpec((D, TF), lambda fo, tb: (0, fo)),
            pl.BlockSpec((D, TF), lambda fo, tb: (0, fo)),
        ],
        out_specs=pl.BlockSpec((BQ, TF), lambda fo, tb: (tb, fo)),
        out_shape=jax.ShapeDtypeStruct((T, F), _bf16),
        scratch_shapes=[pltpu.VMEM((D, TF), _bf16),
                        pltpu.VMEM((D, TF), _bf16)],
        compiler_params=pltpu.CompilerParams(
            dimension_semantics=("arbitrary", "arbitrary")),
    )(x, ws_gate, ws_up)


def _downs_body(h_ref, wd_ref, rg_ref, o_ref, wd16):
    tb = pl.program_id(1)

    @pl.when(tb == 0)
    def _():
        wd16[...] = wd_ref[...].astype(_bf16)

    o_ref[...] = (jnp.dot(h_ref[...], wd16[...], preferred_element_type=_f32)
                  + rg_ref[...])


def _down_shared(h, ws_down, routed):
    return pl.pallas_call(
        _downs_body,
        grid=(ND, NTB),
        in_specs=[
            pl.BlockSpec((BQ, F), lambda dc, tb: (tb, 0)),
            pl.BlockSpec((F, TD), lambda dc, tb: (0, dc)),
            pl.BlockSpec((BQ, TD), lambda dc, tb: (tb, dc)),
        ],
        out_specs=pl.BlockSpec((BQ, TD), lambda dc, tb: (tb, dc)),
        out_shape=jax.ShapeDtypeStruct((T, D), _f32),
        scratch_shapes=[pltpu.VMEM((F, TD), _bf16)],
        compiler_params=pltpu.CompilerParams(
            dimension_semantics=("arbitrary", "arbitrary")),
    )(h, ws_down, routed)


def kernel(hidden_states, W_router, W_gate, W_up, W_down,
           Ws_gate, Ws_up, Ws_down):
    wr_pad = jnp.pad(W_router, ((0, 0), (0, EP - E)))
    eid, pos, score, xs, counts = _router(hidden_states, wr_pad)
    counts_i = counts.astype(_i32)
    slot2, be = _bookkeep(counts_i, eid.reshape(T // TB, TB),
                          pos.reshape(T // TB, TB))
    slot = slot2.reshape(T)
    be_flat = be[0, :NB]

    x_sorted = jnp.zeros((NBQ, D), _f32).at[slot].set(xs)
    h = _up_routed(be_flat, x_sorted, W_gate, W_up)
    out_sorted = _down_routed(be_flat, h, W_down)
    routed = out_sorted[slot]

    hs = _up_shared(hidden_states, Ws_gate, Ws_up)
    return _down_shared(hs, Ws_down, routed)


# speedup vs baseline: 1.0541x; 1.0541x over previous
import functools

import jax
import jax.numpy as jnp
from jax import lax
from jax.experimental import pallas as pl
from jax.experimental.pallas import tpu as pltpu
from jax.experimental.pallas import tpu_sc as plsc

T, D, F, E = 4096, 2048, 2048, 8
EP = 128
TB = 512
BQ = 256
NB = T // BQ + E
NBQ = NB * BQ
TF = 1024
NF = F // TF
TD = 1024
ND = D // TD
BE_NONE = 127
NTB = T // BQ

_f32 = jnp.float32
_bf16 = jnp.bfloat16
_i32 = jnp.int32


def _router_body(x_ref, wr_ref, xs_ref, slot_ref, be_ref,
                 csum_ref, eid_s, pos_s):
    b = pl.program_id(0)
    nph = T // TB

    @pl.when(b == 0)
    def _():
        csum_ref[...] = jnp.zeros_like(csum_ref)

    @pl.when(b < nph)
    def _():
        x = x_ref[...]
        logits = jnp.dot(x, wr_ref[...], preferred_element_type=_f32)
        col = lax.broadcasted_iota(_i32, (TB, EP), 1)
        logits = jnp.where(col < E, logits, -jnp.inf)
        m = jnp.max(logits, axis=1, keepdims=True)
        eid = jnp.min(jnp.where(logits == m, col, EP), axis=1, keepdims=True)
        score = 1.0 / (1.0 + jnp.exp(-m))
        onehot = (col == eid).astype(_f32)
        rio = lax.broadcasted_iota(_i32, (TB, TB), 0)
        cio = lax.broadcasted_iota(_i32, (TB, TB), 1)
        tri = (cio < rio).astype(_f32)
        posmat = jnp.dot(tri, onehot, preferred_element_type=_f32)
        prev = csum_ref[...]
        pos = jnp.sum((posmat + prev) * onehot, axis=1, keepdims=True)
        csum_ref[...] = prev + jnp.sum(onehot, axis=0, keepdims=True)
        eid_s[pl.ds(b * TB, TB), :] = eid
        pos_s[pl.ds(b * TB, TB), :] = pos.astype(_i32)
        xs_ref[...] = x * score

    @pl.when(b >= nph)
    def _():
        tb = b - nph
        cnt = csum_ref[...]
        nbv = jnp.floor((cnt + (BQ - 1)) * (1.0 / BQ))
        lio = lax.broadcasted_iota(_i32, (EP, EP), 0)
        cio2 = lax.broadcasted_iota(_i32, (EP, EP), 1)
        tri_incl = (lio <= cio2).astype(_f32)
        ends = jnp.dot(nbv, tri_incl,
                       preferred_element_type=_f32)
        blkstart = ends - nbv
        eid = eid_s[pl.ds(tb * TB, TB), :]
        pos = pos_s[pl.ds(tb * TB, TB), :]
        col = lax.broadcasted_iota(_i32, (TB, EP), 1)
        onehot = (col == eid).astype(_f32)
        base = jnp.sum(blkstart * onehot, axis=1, keepdims=True)
        slot_ref[...] = (base * BQ).astype(_i32) + pos
        jrow = lax.broadcasted_iota(_i32, (EP, EP), 0)
        endsb = jnp.broadcast_to(ends, (EP, EP))
        colmask = lax.broadcasted_iota(_i32, (EP, EP), 1) < E
        a = ((jrow.astype(_f32) >= endsb) & colmask).astype(_f32)
        beacc = jnp.sum(a, axis=1, keepdims=True).astype(_i32)
        be_ref[...] = jnp.where(beacc >= E, BE_NONE, beacc)


def _router(x, wr_pad):
    nph = T // TB
    return pl.pallas_call(
        _router_body,
        grid=(2 * nph,),
        in_specs=[
            pl.BlockSpec((TB, D), lambda b: (jnp.minimum(b, nph - 1), 0)),
            pl.BlockSpec((D, EP), lambda b: (0, 0)),
        ],
        out_specs=[
            pl.BlockSpec((TB, D), lambda b: (jnp.minimum(b, nph - 1), 0)),
            pl.BlockSpec((TB, 1), lambda b: (jnp.maximum(b - nph, 0), 0)),
            pl.BlockSpec((EP, 1), lambda b: (0, 0)),
        ],
        out_shape=[
            jax.ShapeDtypeStruct((T, D), _f32),
            jax.ShapeDtypeStruct((T, 1), _i32),
            jax.ShapeDtypeStruct((EP, 1), _i32),
        ],
        scratch_shapes=[pltpu.VMEM((1, EP), _f32),
                        pltpu.VMEM((T, 1), _i32),
                        pltpu.VMEM((T, 1), _i32)],
        compiler_params=pltpu.CompilerParams(
            dimension_semantics=("arbitrary",)),
    )(x, wr_pad)


_NC, _NS = 2, 16
_NW = _NC * _NS
_PW = T // _NW
_CS = 32


def _sc_scatter(xs, slot):
    mesh = plsc.VectorSubcoreMesh(core_axis_name="c", subcore_axis_name="s")

    @functools.partial(
        pl.kernel,
        out_type=jax.ShapeDtypeStruct((NBQ, D), _f32),
        mesh=mesh,
        scratch_types=[
            pltpu.VMEM((_CS,), _i32),
            pltpu.VMEM((_CS, D), _f32),
            pltpu.SemaphoreType.DMA,
        ],
    )
    def k(xs_hbm, slot_hbm, out_hbm, idx_v, rows_v, sem):
        wid = lax.axis_index("s") * _NC + lax.axis_index("c")
        base = wid * _PW

        def body(j, carry):
            off = base + j * _CS
            pltpu.sync_copy(slot_hbm.at[pl.ds(off, _CS)], idx_v)
            pltpu.sync_copy(xs_hbm.at[pl.ds(off, _CS)], rows_v)
            pltpu.async_copy(rows_v, out_hbm.at[idx_v], sem).wait()
            return carry

        lax.fori_loop(0, _PW // _CS, body, 0)

    return k(xs, slot)


def _sc_gather(out_sorted, slot):
    mesh = plsc.VectorSubcoreMesh(core_axis_name="c", subcore_axis_name="s")

    @functools.partial(
        pl.kernel,
        out_type=jax.ShapeDtypeStruct((T, D), _f32),
        mesh=mesh,
        scratch_types=[
            pltpu.VMEM((_CS,), _i32),
            pltpu.VMEM((_CS, D), _f32),
            pltpu.SemaphoreType.DMA,
        ],
    )
    def k(src_hbm, slot_hbm, out_hbm, idx_v, rows_v, sem):
        wid = lax.axis_index("s") * _NC + lax.axis_index("c")
        base = wid * _PW

        def body(j, carry):
            off = base + j * _CS
            pltpu.sync_copy(slot_hbm.at[pl.ds(off, _CS)], idx_v)
            pltpu.async_copy(src_hbm.at[idx_v], rows_v, sem).wait()
            pltpu.sync_copy(rows_v, out_hbm.at[pl.ds(off, _CS)])
            return carry

        lax.fori_loop(0, _PW // _CS, body, 0)

    return k(out_sorted, slot)


def _upr_body(be_ref, x_ref, wg_ref, wu_ref, h_ref, wg16, wu16):
    b = pl.program_id(1)
    e = be_ref[b]
    eprev = be_ref[jnp.maximum(b - 1, 0)]

    @pl.when(e != BE_NONE)
    def _():
        @pl.when((b == 0) | (e != eprev))
        def _():
            wg16[...] = wg_ref[0].astype(_bf16)
            wu16[...] = wu_ref[0].astype(_bf16)

        x16 = x_ref[...].astype(_bf16)
        g = jnp.dot(x16, wg16[...], preferred_element_type=_f32)
        u = jnp.dot(x16, wu16[...], preferred_element_type=_f32)
        h = g / (1.0 + jnp.exp(-g)) * u
        h_ref[...] = h.astype(_bf16)


def _up_routed(be, x_sorted, w_gate, w_up):
    grid_spec = pltpu.PrefetchScalarGridSpec(
        num_scalar_prefetch=1,
        grid=(NF, NB),
        in_specs=[
            pl.BlockSpec((BQ, D),
                         lambda fo, b, be: (jnp.where(be[b] == BE_NONE, 0, b),
                                            0)),
            pl.BlockSpec((1, D, TF),
                         lambda fo, b, be: (jnp.minimum(be[b], E - 1), 0, fo)),
            pl.BlockSpec((1, D, TF),
                         lambda fo, b, be: (jnp.minimum(be[b], E - 1), 0, fo)),
        ],
        out_specs=pl.BlockSpec((BQ, TF), lambda fo, b, be: (b, fo)),
        scratch_shapes=[pltpu.VMEM((D, TF), _bf16),
                        pltpu.VMEM((D, TF), _bf16)],
    )
    return pl.pallas_call(
        _upr_body,
        grid_spec=grid_spec,
        out_shape=jax.ShapeDtypeStruct((NBQ, F), _bf16),
        compiler_params=pltpu.CompilerParams(
            dimension_semantics=("arbitrary", "arbitrary")),
    )(be, x_sorted, w_gate, w_up)


def _downr_body(be_ref, h_ref, wd_ref, o_ref, wd16):
    b = pl.program_id(1)
    e = be_ref[b]
    eprev = be_ref[jnp.maximum(b - 1, 0)]

    @pl.when(e != BE_NONE)
    def _():
        @pl.when((b == 0) | (e != eprev))
        def _():
            wd16[...] = wd_ref[0].astype(_bf16)

        o_ref[...] = jnp.dot(h_ref[...], wd16[...],
                             preferred_element_type=_f32)


def _down_routed(be, h, w_down):
    grid_spec = pltpu.PrefetchScalarGridSpec(
        num_scalar_prefetch=1,
        grid=(ND, NB),
        in_specs=[
            pl.BlockSpec((BQ, F),
                         lambda dc, b, be: (jnp.where(be[b] == BE_NONE, 0, b),
                                            0)),
            pl.BlockSpec((1, F, TD),
                         lambda dc, b, be: (jnp.minimum(be[b], E - 1), 0, dc)),
        ],
        out_specs=pl.BlockSpec((BQ, TD), lambda dc, b, be: (b, dc)),
        scratch_shapes=[pltpu.VMEM((F, TD), _bf16)],
    )
    return pl.pallas_call(
        _downr_body,
        grid_spec=grid_spec,
        out_shape=jax.ShapeDtypeStruct((NBQ, D), _f32),
        compiler_params=pltpu.CompilerParams(
            dimension_semantics=("arbitrary", "arbitrary")),
    )(be, h, w_down)


def _ups_body(x_ref, wg_ref, wu_ref, h_ref, wg16, wu16):
    tb = pl.program_id(1)

    @pl.when(tb == 0)
    def _():
        wg16[...] = wg_ref[...].astype(_bf16)
        wu16[...] = wu_ref[...].astype(_bf16)

    x16 = x_ref[...].astype(_bf16)
    g = jnp.dot(x16, wg16[...], preferred_element_type=_f32)
    u = jnp.dot(x16, wu16[...], preferred_element_type=_f32)
    h = g / (1.0 + jnp.exp(-g)) * u
    h_ref[...] = h.astype(_bf16)


def _up_shared(x, ws_gate, ws_up):
    return pl.pallas_call(
        _ups_body,
        grid=(NF, NTB),
        in_specs=[
            pl.BlockSpec((BQ, D), lambda fo, tb: (tb, 0)),
            pl.BlockSpec((D, TF), lambda fo, tb: (0, fo)),
            pl.BlockSpec((D, TF), lambda fo, tb: (0, fo)),
        ],
        out_specs=pl.BlockSpec((BQ, TF), lambda fo, tb: (tb, fo)),
        out_shape=jax.ShapeDtypeStruct((T, F), _bf16),
        scratch_shapes=[pltpu.VMEM((D, TF), _bf16),
                        pltpu.VMEM((D, TF), _bf16)],
        compiler_params=pltpu.CompilerParams(
            dimension_semantics=("arbitrary", "arbitrary")),
    )(x, ws_gate, ws_up)


def _downs_body(h_ref, wd_ref, rg_ref, o_ref, wd16):
    tb = pl.program_id(1)

    @pl.when(tb == 0)
    def _():
        wd16[...] = wd_ref[...].astype(_bf16)

    o_ref[...] = (jnp.dot(h_ref[...], wd16[...], preferred_element_type=_f32)
                  + rg_ref[...])


def _down_shared(h, ws_down, routed):
    return pl.pallas_call(
        _downs_body,
        grid=(ND, NTB),
        in_specs=[
            pl.BlockSpec((BQ, F), lambda dc, tb: (tb, 0)),
            pl.BlockSpec((F, TD), lambda dc, tb: (0, dc)),
            pl.BlockSpec((BQ, TD), lambda dc, tb: (tb, dc)),
        ],
        out_specs=pl.BlockSpec((BQ, TD), lambda dc, tb: (tb, dc)),
        out_shape=jax.ShapeDtypeStruct((T, D), _f32),
        scratch_shapes=[pltpu.VMEM((F, TD), _bf16)],
        compiler_params=pltpu.CompilerParams(
            dimension_semantics=("arbitrary", "arbitrary")),
    )(h, ws_down, routed)


def kernel(hidden_states, W_router, W_gate, W_up, W_down,
           Ws_gate, Ws_up, Ws_down):
    wr_pad = jnp.pad(W_router, ((0, 0), (0, EP - E)))
    xs, slot2, be = _router(hidden_states, wr_pad)
    slot = slot2.reshape(T)
    be_flat = be[:NB, 0]

    x_sorted = _sc_scatter(xs, slot)
    h = _up_routed(be_flat, x_sorted, W_gate, W_up)
    out_sorted = _down_routed(be_flat, h, W_down)
    routed = _sc_gather(out_sorted, slot)

    hs = _up_shared(hidden_states, Ws_gate, Ws_up)
    return _down_shared(hs, Ws_down, routed)


# speedup vs baseline: 1.1100x; 1.0530x over previous
import functools

import jax
import jax.numpy as jnp
from jax import lax
from jax.experimental import pallas as pl
from jax.experimental.pallas import tpu as pltpu
from jax.experimental.pallas import tpu_sc as plsc

T, D, F, E = 4096, 2048, 2048, 8
EP = 128
TB = 512
BQ = 256
NB = T // BQ + E
NBQ = NB * BQ
TF = 1024
NF = F // TF
TD = 2048
ND = D // TD
BE_NONE = 127
NTB = T // BQ

_f32 = jnp.float32
_bf16 = jnp.bfloat16
_i32 = jnp.int32


def _router_body(x_ref, wr_ref, xs_ref, slot_ref, be_ref,
                 csum_ref, eid_s, pos_s):
    b = pl.program_id(0)
    nph = T // TB

    @pl.when(b == 0)
    def _():
        csum_ref[...] = jnp.zeros_like(csum_ref)

    @pl.when(b < nph)
    def _():
        x = x_ref[...]
        logits = jnp.dot(x, wr_ref[...], preferred_element_type=_f32)
        col = lax.broadcasted_iota(_i32, (TB, EP), 1)
        logits = jnp.where(col < E, logits, -jnp.inf)
        m = jnp.max(logits, axis=1, keepdims=True)
        eid = jnp.min(jnp.where(logits == m, col, EP), axis=1, keepdims=True)
        score = 1.0 / (1.0 + jnp.exp(-m))
        onehot = (col == eid).astype(_f32)
        rio = lax.broadcasted_iota(_i32, (TB, TB), 0)
        cio = lax.broadcasted_iota(_i32, (TB, TB), 1)
        tri = (cio < rio).astype(_f32)
        posmat = jnp.dot(tri, onehot, preferred_element_type=_f32)
        prev = csum_ref[...]
        pos = jnp.sum((posmat + prev) * onehot, axis=1, keepdims=True)
        csum_ref[...] = prev + jnp.sum(onehot, axis=0, keepdims=True)
        eid_s[pl.ds(b * TB, TB), :] = eid
        pos_s[pl.ds(b * TB, TB), :] = pos.astype(_i32)
        xs_ref[...] = x * score

    @pl.when(b >= nph)
    def _():
        tb = b - nph
        cnt = csum_ref[...]
        nbv = jnp.floor((cnt + (BQ - 1)) * (1.0 / BQ))
        lio = lax.broadcasted_iota(_i32, (EP, EP), 0)
        cio2 = lax.broadcasted_iota(_i32, (EP, EP), 1)
        tri_incl = (lio <= cio2).astype(_f32)
        ends = jnp.dot(nbv, tri_incl,
                       preferred_element_type=_f32)
        blkstart = ends - nbv
        eid = eid_s[pl.ds(tb * TB, TB), :]
        pos = pos_s[pl.ds(tb * TB, TB), :]
        col = lax.broadcasted_iota(_i32, (TB, EP), 1)
        onehot = (col == eid).astype(_f32)
        base = jnp.sum(blkstart * onehot, axis=1, keepdims=True)
        slot_ref[...] = (base * BQ).astype(_i32) + pos
        jrow = lax.broadcasted_iota(_i32, (EP, EP), 0)
        endsb = jnp.broadcast_to(ends, (EP, EP))
        colmask = lax.broadcasted_iota(_i32, (EP, EP), 1) < E
        a = ((jrow.astype(_f32) >= endsb) & colmask).astype(_f32)
        beacc = jnp.sum(a, axis=1, keepdims=True).astype(_i32)
        be_ref[...] = jnp.where(beacc >= E, BE_NONE, beacc)


def _router(x, wr_pad):
    nph = T // TB
    return pl.pallas_call(
        _router_body,
        grid=(2 * nph,),
        in_specs=[
            pl.BlockSpec((TB, D), lambda b: (jnp.minimum(b, nph - 1), 0)),
            pl.BlockSpec((D, EP), lambda b: (0, 0)),
        ],
        out_specs=[
            pl.BlockSpec((TB, D), lambda b: (jnp.minimum(b, nph - 1), 0)),
            pl.BlockSpec((TB, 1), lambda b: (jnp.maximum(b - nph, 0), 0)),
            pl.BlockSpec((EP, 1), lambda b: (0, 0)),
        ],
        out_shape=[
            jax.ShapeDtypeStruct((T, D), _f32),
            jax.ShapeDtypeStruct((T, 1), _i32),
            jax.ShapeDtypeStruct((EP, 1), _i32),
        ],
        scratch_shapes=[pltpu.VMEM((1, EP), _f32),
                        pltpu.VMEM((T, 1), _i32),
                        pltpu.VMEM((T, 1), _i32)],
        compiler_params=pltpu.CompilerParams(
            dimension_semantics=("arbitrary",)),
    )(x, wr_pad)


_NC, _NS = 2, 16
_NW = _NC * _NS
_PW = T // _NW
_CS = 32


def _sc_scatter(xs, slot):
    mesh = plsc.VectorSubcoreMesh(core_axis_name="c", subcore_axis_name="s")

    @functools.partial(
        pl.kernel,
        out_type=jax.ShapeDtypeStruct((NBQ, D), _f32),
        mesh=mesh,
        scratch_types=[
            pltpu.VMEM((_CS,), _i32),
            pltpu.VMEM((_CS, D), _f32),
            pltpu.SemaphoreType.DMA,
        ],
    )
    def k(xs_hbm, slot_hbm, out_hbm, idx_v, rows_v, sem):
        wid = lax.axis_index("s") * _NC + lax.axis_index("c")
        base = wid * _PW

        def body(j, carry):
            off = base + j * _CS
            pltpu.sync_copy(slot_hbm.at[pl.ds(off, _CS)], idx_v)
            pltpu.sync_copy(xs_hbm.at[pl.ds(off, _CS)], rows_v)
            pltpu.async_copy(rows_v, out_hbm.at[idx_v], sem).wait()
            return carry

        lax.fori_loop(0, _PW // _CS, body, 0)

    return k(xs, slot)


def _sc_gather(out_sorted, slot):
    mesh = plsc.VectorSubcoreMesh(core_axis_name="c", subcore_axis_name="s")

    @functools.partial(
        pl.kernel,
        out_type=jax.ShapeDtypeStruct((T, D), _f32),
        mesh=mesh,
        scratch_types=[
            pltpu.VMEM((_CS,), _i32),
            pltpu.VMEM((_CS, D), _f32),
            pltpu.SemaphoreType.DMA,
        ],
    )
    def k(src_hbm, slot_hbm, out_hbm, idx_v, rows_v, sem):
        wid = lax.axis_index("s") * _NC + lax.axis_index("c")
        base = wid * _PW

        def body(j, carry):
            off = base + j * _CS
            pltpu.sync_copy(slot_hbm.at[pl.ds(off, _CS)], idx_v)
            pltpu.async_copy(src_hbm.at[idx_v], rows_v, sem).wait()
            pltpu.sync_copy(rows_v, out_hbm.at[pl.ds(off, _CS)])
            return carry

        lax.fori_loop(0, _PW // _CS, body, 0)

    return k(out_sorted, slot)


def _upr_body(be_ref, x_ref, wg_ref, wu_ref, h_ref, wg16, wu16):
    b = pl.program_id(1)
    e = be_ref[b]
    eprev = be_ref[jnp.maximum(b - 1, 0)]

    @pl.when(e != BE_NONE)
    def _():
        @pl.when((b == 0) | (e != eprev))
        def _():
            wg16[...] = wg_ref[0].astype(_bf16)
            wu16[...] = wu_ref[0].astype(_bf16)

        x16 = x_ref[...].astype(_bf16)
        g = jnp.dot(x16, wg16[...], preferred_element_type=_f32)
        u = jnp.dot(x16, wu16[...], preferred_element_type=_f32)
        h = g / (1.0 + jnp.exp(-g)) * u
        h_ref[...] = h.astype(_bf16)


def _up_routed(be, x_sorted, w_gate, w_up):
    grid_spec = pltpu.PrefetchScalarGridSpec(
        num_scalar_prefetch=1,
        grid=(NF, NB),
        in_specs=[
            pl.BlockSpec((BQ, D),
                         lambda fo, b, be: (jnp.where(be[b] == BE_NONE, 0, b),
                                            0)),
            pl.BlockSpec((1, D, TF),
                         lambda fo, b, be: (jnp.minimum(be[b], E - 1), 0, fo)),
            pl.BlockSpec((1, D, TF),
                         lambda fo, b, be: (jnp.minimum(be[b], E - 1), 0, fo)),
        ],
        out_specs=pl.BlockSpec((BQ, TF), lambda fo, b, be: (b, fo)),
        scratch_shapes=[pltpu.VMEM((D, TF), _bf16),
                        pltpu.VMEM((D, TF), _bf16)],
    )
    return pl.pallas_call(
        _upr_body,
        grid_spec=grid_spec,
        out_shape=jax.ShapeDtypeStruct((NBQ, F), _bf16),
        compiler_params=pltpu.CompilerParams(
            dimension_semantics=("arbitrary", "arbitrary")),
    )(be, x_sorted, w_gate, w_up)


def _downr_body(be_ref, h_ref, wd_ref, o_ref, wd16):
    b = pl.program_id(1)
    e = be_ref[b]
    eprev = be_ref[jnp.maximum(b - 1, 0)]

    @pl.when(e != BE_NONE)
    def _():
        @pl.when((b == 0) | (e != eprev))
        def _():
            wd16[...] = wd_ref[0].astype(_bf16)

        o_ref[...] = jnp.dot(h_ref[...], wd16[...],
                             preferred_element_type=_f32)


def _down_routed(be, h, w_down):
    grid_spec = pltpu.PrefetchScalarGridSpec(
        num_scalar_prefetch=1,
        grid=(ND, NB),
        in_specs=[
            pl.BlockSpec((BQ, F),
                         lambda dc, b, be: (jnp.where(be[b] == BE_NONE, 0, b),
                                            0)),
            pl.BlockSpec((1, F, TD),
                         lambda dc, b, be: (jnp.minimum(be[b], E - 1), 0, dc)),
        ],
        out_specs=pl.BlockSpec((BQ, TD), lambda dc, b, be: (b, dc)),
        scratch_shapes=[pltpu.VMEM((F, TD), _bf16)],
    )
    return pl.pallas_call(
        _downr_body,
        grid_spec=grid_spec,
        out_shape=jax.ShapeDtypeStruct((NBQ, D), _f32),
        compiler_params=pltpu.CompilerParams(
            dimension_semantics=("arbitrary", "arbitrary")),
    )(be, h, w_down)


def _ups_body(x_ref, wg_ref, wu_ref, h_ref, wg16, wu16):
    tb = pl.program_id(1)

    @pl.when(tb == 0)
    def _():
        wg16[...] = wg_ref[...].astype(_bf16)
        wu16[...] = wu_ref[...].astype(_bf16)

    x16 = x_ref[...].astype(_bf16)
    g = jnp.dot(x16, wg16[...], preferred_element_type=_f32)
    u = jnp.dot(x16, wu16[...], preferred_element_type=_f32)
    h = g / (1.0 + jnp.exp(-g)) * u
    h_ref[...] = h.astype(_bf16)


def _up_shared(x, ws_gate, ws_up):
    return pl.pallas_call(
        _ups_body,
        grid=(NF, NTB),
        in_specs=[
            pl.BlockSpec((BQ, D), lambda fo, tb: (tb, 0)),
            pl.BlockSpec((D, TF), lambda fo, tb: (0, fo)),
            pl.BlockSpec((D, TF), lambda fo, tb: (0, fo)),
        ],
        out_specs=pl.BlockSpec((BQ, TF), lambda fo, tb: (tb, fo)),
        out_shape=jax.ShapeDtypeStruct((T, F), _bf16),
        scratch_shapes=[pltpu.VMEM((D, TF), _bf16),
                        pltpu.VMEM((D, TF), _bf16)],
        compiler_params=pltpu.CompilerParams(
            dimension_semantics=("arbitrary", "arbitrary")),
    )(x, ws_gate, ws_up)


def _downs_body(h_ref, wd_ref, rg_ref, o_ref, wd16):
    tb = pl.program_id(1)

    @pl.when(tb == 0)
    def _():
        wd16[...] = wd_ref[...].astype(_bf16)

    o_ref[...] = (jnp.dot(h_ref[...], wd16[...], preferred_element_type=_f32)
                  + rg_ref[...])


def _down_shared(h, ws_down, routed):
    return pl.pallas_call(
        _downs_body,
        grid=(ND, NTB),
        in_specs=[
            pl.BlockSpec((BQ, F), lambda dc, tb: (tb, 0)),
            pl.BlockSpec((F, TD), lambda dc, tb: (0, dc)),
            pl.BlockSpec((BQ, TD), lambda dc, tb: (tb, dc)),
        ],
        out_specs=pl.BlockSpec((BQ, TD), lambda dc, tb: (tb, dc)),
        out_shape=jax.ShapeDtypeStruct((T, D), _f32),
        scratch_shapes=[pltpu.VMEM((F, TD), _bf16)],
        compiler_params=pltpu.CompilerParams(
            dimension_semantics=("arbitrary", "arbitrary")),
    )(h, ws_down, routed)


def kernel(hidden_states, W_router, W_gate, W_up, W_down,
           Ws_gate, Ws_up, Ws_down):
    wr_pad = jnp.pad(W_router, ((0, 0), (0, EP - E)))
    xs, slot2, be = _router(hidden_states, wr_pad)
    slot = slot2.reshape(T)
    be_flat = be[:NB, 0]

    x_sorted = _sc_scatter(xs, slot)
    h = _up_routed(be_flat, x_sorted, W_gate, W_up)
    out_sorted = _down_routed(be_flat, h, W_down)
    routed = _sc_gather(out_sorted, slot)

    hs = _up_shared(hidden_states, Ws_gate, Ws_up)
    return _down_shared(hs, Ws_down, routed)


# speedup vs baseline: 1.1125x; 1.0023x over previous
import functools

import jax
import jax.numpy as jnp
from jax import lax
from jax.experimental import pallas as pl
from jax.experimental.pallas import tpu as pltpu
from jax.experimental.pallas import tpu_sc as plsc

T, D, F, E = 4096, 2048, 2048, 8
EP = 128
TB = 512
BQ = 256
NB = T // BQ + E
NBQ = NB * BQ
TF = 1024
NF = F // TF
TD = 2048
ND = D // TD
BE_NONE = 127
NTB = T // BQ

_f32 = jnp.float32
_bf16 = jnp.bfloat16
_i32 = jnp.int32


def _router_body(x_ref, wr_ref, sb_ref, slot_ref, be_ref,
                 csum_ref, eid_s, pos_s):
    b = pl.program_id(0)
    nph = T // TB

    @pl.when(b == 0)
    def _():
        csum_ref[...] = jnp.zeros_like(csum_ref)

    @pl.when(b < nph)
    def _():
        x = x_ref[...]
        logits = jnp.dot(x, wr_ref[...], preferred_element_type=_f32)
        col = lax.broadcasted_iota(_i32, (TB, EP), 1)
        logits = jnp.where(col < E, logits, -jnp.inf)
        m = jnp.max(logits, axis=1, keepdims=True)
        eid = jnp.min(jnp.where(logits == m, col, EP), axis=1, keepdims=True)
        score = 1.0 / (1.0 + jnp.exp(-m))
        onehot = (col == eid).astype(_f32)
        rio = lax.broadcasted_iota(_i32, (TB, TB), 0)
        cio = lax.broadcasted_iota(_i32, (TB, TB), 1)
        tri = (cio < rio).astype(_f32)
        posmat = jnp.dot(tri, onehot, preferred_element_type=_f32)
        prev = csum_ref[...]
        pos = jnp.sum((posmat + prev) * onehot, axis=1, keepdims=True)
        csum_ref[...] = prev + jnp.sum(onehot, axis=0, keepdims=True)
        eid_s[pl.ds(b * TB, TB), :] = eid
        pos_s[pl.ds(b * TB, TB), :] = pos.astype(_i32)
        sb_ref[...] = jnp.broadcast_to(score, (TB, EP))

    @pl.when(b >= nph)
    def _():
        tb = b - nph
        cnt = csum_ref[...]
        nbv = jnp.floor((cnt + (BQ - 1)) * (1.0 / BQ))
        lio = lax.broadcasted_iota(_i32, (EP, EP), 0)
        cio2 = lax.broadcasted_iota(_i32, (EP, EP), 1)
        tri_incl = (lio <= cio2).astype(_f32)
        ends = jnp.dot(nbv, tri_incl,
                       preferred_element_type=_f32)
        blkstart = ends - nbv
        eid = eid_s[pl.ds(tb * TB, TB), :]
        pos = pos_s[pl.ds(tb * TB, TB), :]
        col = lax.broadcasted_iota(_i32, (TB, EP), 1)
        onehot = (col == eid).astype(_f32)
        base = jnp.sum(blkstart * onehot, axis=1, keepdims=True)
        slot_ref[...] = (base * BQ).astype(_i32) + pos
        jrow = lax.broadcasted_iota(_i32, (EP, EP), 0)
        endsb = jnp.broadcast_to(ends, (EP, EP))
        colmask = lax.broadcasted_iota(_i32, (EP, EP), 1) < E
        a = ((jrow.astype(_f32) >= endsb) & colmask).astype(_f32)
        beacc = jnp.sum(a, axis=1, keepdims=True).astype(_i32)
        be_ref[...] = jnp.where(beacc >= E, BE_NONE, beacc)


def _router(x, wr_pad):
    nph = T // TB
    return pl.pallas_call(
        _router_body,
        grid=(2 * nph,),
        in_specs=[
            pl.BlockSpec((TB, D), lambda b: (jnp.minimum(b, nph - 1), 0)),
            pl.BlockSpec((D, EP), lambda b: (0, 0)),
        ],
        out_specs=[
            pl.BlockSpec((TB, EP), lambda b: (jnp.minimum(b, nph - 1), 0)),
            pl.BlockSpec((TB, 1), lambda b: (jnp.maximum(b - nph, 0), 0)),
            pl.BlockSpec((EP, 1), lambda b: (0, 0)),
        ],
        out_shape=[
            jax.ShapeDtypeStruct((T, EP), _f32),
            jax.ShapeDtypeStruct((T, 1), _i32),
            jax.ShapeDtypeStruct((EP, 1), _i32),
        ],
        scratch_shapes=[pltpu.VMEM((1, EP), _f32),
                        pltpu.VMEM((T, 1), _i32),
                        pltpu.VMEM((T, 1), _i32)],
        compiler_params=pltpu.CompilerParams(
            dimension_semantics=("arbitrary",)),
    )(x, wr_pad)


_NC, _NS = 2, 16
_NW = _NC * _NS
_PW = T // _NW
_CS = 32


def _sc_scatter(x, score_b, slot):
    mesh = plsc.VectorSubcoreMesh(core_axis_name="c", subcore_axis_name="s")

    @functools.partial(
        pl.kernel,
        out_type=(jax.ShapeDtypeStruct((NBQ, D), _f32),
                  jax.ShapeDtypeStruct((NBQ, EP), _f32)),
        mesh=mesh,
        scratch_types=[
            pltpu.VMEM((_CS,), _i32),
            pltpu.VMEM((_CS, D), _f32),
            pltpu.VMEM((_CS, EP), _f32),
            pltpu.SemaphoreType.DMA,
        ],
    )
    def k(x_hbm, sb_hbm, slot_hbm, out_hbm, ss_hbm, idx_v, rows_v, srow_v,
          sem):
        wid = lax.axis_index("s") * _NC + lax.axis_index("c")
        base = wid * _PW

        def body(j, carry):
            off = base + j * _CS
            pltpu.sync_copy(slot_hbm.at[pl.ds(off, _CS)], idx_v)
            pltpu.sync_copy(x_hbm.at[pl.ds(off, _CS)], rows_v)
            pltpu.sync_copy(sb_hbm.at[pl.ds(off, _CS)], srow_v)
            pltpu.async_copy(rows_v, out_hbm.at[idx_v], sem).wait()
            pltpu.async_copy(srow_v, ss_hbm.at[idx_v], sem).wait()
            return carry

        lax.fori_loop(0, _PW // _CS, body, 0)

    return k(x, score_b, slot)


def _sc_gather(out_sorted, slot):
    mesh = plsc.VectorSubcoreMesh(core_axis_name="c", subcore_axis_name="s")

    @functools.partial(
        pl.kernel,
        out_type=jax.ShapeDtypeStruct((T, D), _f32),
        mesh=mesh,
        scratch_types=[
            pltpu.VMEM((_CS,), _i32),
            pltpu.VMEM((_CS, D), _f32),
            pltpu.SemaphoreType.DMA,
        ],
    )
    def k(src_hbm, slot_hbm, out_hbm, idx_v, rows_v, sem):
        wid = lax.axis_index("s") * _NC + lax.axis_index("c")
        base = wid * _PW

        def body(j, carry):
            off = base + j * _CS
            pltpu.sync_copy(slot_hbm.at[pl.ds(off, _CS)], idx_v)
            pltpu.async_copy(src_hbm.at[idx_v], rows_v, sem).wait()
            pltpu.sync_copy(rows_v, out_hbm.at[pl.ds(off, _CS)])
            return carry

        lax.fori_loop(0, _PW // _CS, body, 0)

    return k(out_sorted, slot)


def _upr_body(be_ref, x_ref, ss_ref, wg_ref, wu_ref, h_ref, wg16, wu16):
    b = pl.program_id(1)
    e = be_ref[b]
    eprev = be_ref[jnp.maximum(b - 1, 0)]

    @pl.when(e != BE_NONE)
    def _():
        @pl.when((b == 0) | (e != eprev))
        def _():
            wg16[...] = wg_ref[0].astype(_bf16)
            wu16[...] = wu_ref[0].astype(_bf16)

        x16 = (x_ref[...] * ss_ref[:, :1]).astype(_bf16)
        g = jnp.dot(x16, wg16[...], preferred_element_type=_f32)
        u = jnp.dot(x16, wu16[...], preferred_element_type=_f32)
        h = g / (1.0 + jnp.exp(-g)) * u
        h_ref[...] = h.astype(_bf16)


def _up_routed(be, x_sorted, ssort, w_gate, w_up):
    grid_spec = pltpu.PrefetchScalarGridSpec(
        num_scalar_prefetch=1,
        grid=(NF, NB),
        in_specs=[
            pl.BlockSpec((BQ, D),
                         lambda fo, b, be: (jnp.where(be[b] == BE_NONE, 0, b),
                                            0)),
            pl.BlockSpec((BQ, EP),
                         lambda fo, b, be: (jnp.where(be[b] == BE_NONE, 0, b),
                                            0)),
            pl.BlockSpec((1, D, TF),
                         lambda fo, b, be: (jnp.minimum(be[b], E - 1), 0, fo)),
            pl.BlockSpec((1, D, TF),
                         lambda fo, b, be: (jnp.minimum(be[b], E - 1), 0, fo)),
        ],
        out_specs=pl.BlockSpec((BQ, TF), lambda fo, b, be: (b, fo)),
        scratch_shapes=[pltpu.VMEM((D, TF), _bf16),
                        pltpu.VMEM((D, TF), _bf16)],
    )
    return pl.pallas_call(
        _upr_body,
        grid_spec=grid_spec,
        out_shape=jax.ShapeDtypeStruct((NBQ, F), _bf16),
        compiler_params=pltpu.CompilerParams(
            dimension_semantics=("arbitrary", "arbitrary")),
    )(be, x_sorted, ssort, w_gate, w_up)


def _downr_body(be_ref, h_ref, wd_ref, o_ref, wd16):
    b = pl.program_id(1)
    e = be_ref[b]
    eprev = be_ref[jnp.maximum(b - 1, 0)]

    @pl.when(e != BE_NONE)
    def _():
        @pl.when((b == 0) | (e != eprev))
        def _():
            wd16[...] = wd_ref[0].astype(_bf16)

        o_ref[...] = jnp.dot(h_ref[...], wd16[...],
                             preferred_element_type=_f32)


def _down_routed(be, h, w_down):
    grid_spec = pltpu.PrefetchScalarGridSpec(
        num_scalar_prefetch=1,
        grid=(ND, NB),
        in_specs=[
            pl.BlockSpec((BQ, F),
                         lambda dc, b, be: (jnp.where(be[b] == BE_NONE, 0, b),
                                            0)),
            pl.BlockSpec((1, F, TD),
                         lambda dc, b, be: (jnp.minimum(be[b], E - 1), 0, dc)),
        ],
        out_specs=pl.BlockSpec((BQ, TD), lambda dc, b, be: (b, dc)),
        scratch_shapes=[pltpu.VMEM((F, TD), _bf16)],
    )
    return pl.pallas_call(
        _downr_body,
        grid_spec=grid_spec,
        out_shape=jax.ShapeDtypeStruct((NBQ, D), _f32),
        compiler_params=pltpu.CompilerParams(
            dimension_semantics=("arbitrary", "arbitrary")),
    )(be, h, w_down)


def _ups_body(x_ref, wg_ref, wu_ref, h_ref, wg16, wu16):
    tb = pl.program_id(1)

    @pl.when(tb == 0)
    def _():
        wg16[...] = wg_ref[...].astype(_bf16)
        wu16[...] = wu_ref[...].astype(_bf16)

    x16 = x_ref[...].astype(_bf16)
    g = jnp.dot(x16, wg16[...], preferred_element_type=_f32)
    u = jnp.dot(x16, wu16[...], preferred_element_type=_f32)
    h = g / (1.0 + jnp.exp(-g)) * u
    h_ref[...] = h.astype(_bf16)


def _up_shared(x, ws_gate, ws_up):
    return pl.pallas_call(
        _ups_body,
        grid=(NF, NTB),
        in_specs=[
            pl.BlockSpec((BQ, D), lambda fo, tb: (tb, 0)),
            pl.BlockSpec((D, TF), lambda fo, tb: (0, fo)),
            pl.BlockSpec((D, TF), lambda fo, tb: (0, fo)),
        ],
        out_specs=pl.BlockSpec((BQ, TF), lambda fo, tb: (tb, fo)),
        out_shape=jax.ShapeDtypeStruct((T, F), _bf16),
        scratch_shapes=[pltpu.VMEM((D, TF), _bf16),
                        pltpu.VMEM((D, TF), _bf16)],
        compiler_params=pltpu.CompilerParams(
            dimension_semantics=("arbitrary", "arbitrary")),
    )(x, ws_gate, ws_up)


def _downs_body(h_ref, wd_ref, rg_ref, o_ref, wd16):
    tb = pl.program_id(1)

    @pl.when(tb == 0)
    def _():
        wd16[...] = wd_ref[...].astype(_bf16)

    o_ref[...] = (jnp.dot(h_ref[...], wd16[...], preferred_element_type=_f32)
                  + rg_ref[...])


def _down_shared(h, ws_down, routed):
    return pl.pallas_call(
        _downs_body,
        grid=(ND, NTB),
        in_specs=[
            pl.BlockSpec((BQ, F), lambda dc, tb: (tb, 0)),
            pl.BlockSpec((F, TD), lambda dc, tb: (0, dc)),
            pl.BlockSpec((BQ, TD), lambda dc, tb: (tb, dc)),
        ],
        out_specs=pl.BlockSpec((BQ, TD), lambda dc, tb: (tb, dc)),
        out_shape=jax.ShapeDtypeStruct((T, D), _f32),
        scratch_shapes=[pltpu.VMEM((F, TD), _bf16)],
        compiler_params=pltpu.CompilerParams(
            dimension_semantics=("arbitrary", "arbitrary")),
    )(h, ws_down, routed)


def kernel(hidden_states, W_router, W_gate, W_up, W_down,
           Ws_gate, Ws_up, Ws_down):
    wr_pad = jnp.pad(W_router, ((0, 0), (0, EP - E)))
    score_b, slot2, be = _router(hidden_states, wr_pad)
    slot = slot2.reshape(T)
    be_flat = be[:NB, 0]

    x_sorted, ssort = _sc_scatter(hidden_states, score_b, slot)
    h = _up_routed(be_flat, x_sorted, ssort, W_gate, W_up)
    out_sorted = _down_routed(be_flat, h, W_down)
    routed = _sc_gather(out_sorted, slot)

    hs = _up_shared(hidden_states, Ws_gate, Ws_up)
    return _down_shared(hs, Ws_down, routed)


# speedup vs baseline: 1.1184x; 1.0053x over previous
import functools

import jax
import jax.numpy as jnp
from jax import lax
from jax.experimental import pallas as pl
from jax.experimental.pallas import tpu as pltpu
from jax.experimental.pallas import tpu_sc as plsc

T, D, F, E = 4096, 2048, 2048, 8
EP = 128
TB = 512
BQ = 256
NB = T // BQ + E
NBQ = NB * BQ
TF = 1024
NF = F // TF
TD = 2048
ND = D // TD
BE_NONE = 127
NTB = T // BQ

_f32 = jnp.float32
_bf16 = jnp.bfloat16
_i32 = jnp.int32


def _router_body(x_ref, wr_ref, sb_ref, slot_ref, be_ref,
                 csum_ref, eid_s, pos_s):
    b = pl.program_id(0)
    nph = T // TB

    @pl.when(b == 0)
    def _():
        csum_ref[...] = jnp.zeros_like(csum_ref)

    @pl.when(b < nph)
    def _():
        x = x_ref[...]
        logits = jnp.dot(x, wr_ref[...], preferred_element_type=_f32)
        col = lax.broadcasted_iota(_i32, (TB, EP), 1)
        logits = jnp.where(col < E, logits, -jnp.inf)
        m = jnp.max(logits, axis=1, keepdims=True)
        eid = jnp.min(jnp.where(logits == m, col, EP), axis=1, keepdims=True)
        score = 1.0 / (1.0 + jnp.exp(-m))
        onehot = (col == eid).astype(_f32)
        rio = lax.broadcasted_iota(_i32, (TB, TB), 0)
        cio = lax.broadcasted_iota(_i32, (TB, TB), 1)
        tri = (cio < rio).astype(_f32)
        posmat = jnp.dot(tri, onehot, preferred_element_type=_f32)
        prev = csum_ref[...]
        pos = jnp.sum((posmat + prev) * onehot, axis=1, keepdims=True)
        csum_ref[...] = prev + jnp.sum(onehot, axis=0, keepdims=True)
        eid_s[pl.ds(b * TB, TB), :] = eid
        pos_s[pl.ds(b * TB, TB), :] = pos.astype(_i32)
        sb_ref[...] = jnp.broadcast_to(score, (TB, EP))

    @pl.when(b >= nph)
    def _():
        tb = b - nph
        cnt = csum_ref[...]
        nbv = jnp.floor((cnt + (BQ - 1)) * (1.0 / BQ))
        lio = lax.broadcasted_iota(_i32, (EP, EP), 0)
        cio2 = lax.broadcasted_iota(_i32, (EP, EP), 1)
        tri_incl = (lio <= cio2).astype(_f32)
        ends = jnp.dot(nbv, tri_incl,
                       preferred_element_type=_f32)
        blkstart = ends - nbv
        eid = eid_s[pl.ds(tb * TB, TB), :]
        pos = pos_s[pl.ds(tb * TB, TB), :]
        col = lax.broadcasted_iota(_i32, (TB, EP), 1)
        onehot = (col == eid).astype(_f32)
        base = jnp.sum(blkstart * onehot, axis=1, keepdims=True)
        slot_ref[...] = (base * BQ).astype(_i32) + pos
        jrow = lax.broadcasted_iota(_i32, (EP, EP), 0)
        endsb = jnp.broadcast_to(ends, (EP, EP))
        colmask = lax.broadcasted_iota(_i32, (EP, EP), 1) < E
        a = ((jrow.astype(_f32) >= endsb) & colmask).astype(_f32)
        beacc = jnp.sum(a, axis=1, keepdims=True).astype(_i32)
        be_ref[...] = jnp.where(beacc >= E, BE_NONE, beacc)


def _router(x, wr_pad):
    nph = T // TB
    return pl.pallas_call(
        _router_body,
        grid=(2 * nph,),
        in_specs=[
            pl.BlockSpec((TB, D), lambda b: (jnp.minimum(b, nph - 1), 0)),
            pl.BlockSpec((D, EP), lambda b: (0, 0)),
        ],
        out_specs=[
            pl.BlockSpec((TB, EP), lambda b: (jnp.minimum(b, nph - 1), 0)),
            pl.BlockSpec((TB, 1), lambda b: (jnp.maximum(b - nph, 0), 0)),
            pl.BlockSpec((EP, 1), lambda b: (0, 0)),
        ],
        out_shape=[
            jax.ShapeDtypeStruct((T, EP), _f32),
            jax.ShapeDtypeStruct((T, 1), _i32),
            jax.ShapeDtypeStruct((EP, 1), _i32),
        ],
        scratch_shapes=[pltpu.VMEM((1, EP), _f32),
                        pltpu.VMEM((T, 1), _i32),
                        pltpu.VMEM((T, 1), _i32)],
        compiler_params=pltpu.CompilerParams(
            dimension_semantics=("arbitrary",)),
    )(x, wr_pad)


_NC, _NS = 2, 16
_NW = _NC * _NS
_PW = T // _NW
_CS = 16
_NCH = _PW // _CS


def _sc_scatter(x, score_b, slot):
    mesh = plsc.VectorSubcoreMesh(core_axis_name="c", subcore_axis_name="s")

    @functools.partial(
        pl.kernel,
        out_type=(jax.ShapeDtypeStruct((NBQ, D), _f32),
                  jax.ShapeDtypeStruct((NBQ, EP), _f32)),
        mesh=mesh,
        scratch_types=[
            pltpu.VMEM((2, _CS), _i32),
            pltpu.VMEM((2, _CS, D), _f32),
            pltpu.VMEM((2, _CS, EP), _f32),
            pltpu.SemaphoreType.DMA,
            pltpu.SemaphoreType.DMA,
        ],
    )
    def k(x_hbm, sb_hbm, slot_hbm, out_hbm, ss_hbm, idx_v, rows_v, srow_v,
          sem_ld, sem_st):
        wid = lax.axis_index("s") * _NC + lax.axis_index("c")
        base = wid * _PW

        def loads(j, bsel):
            off = base + j * _CS
            return (
                pltpu.async_copy(slot_hbm.at[pl.ds(off, _CS)],
                                 idx_v.at[bsel], sem_ld),
                pltpu.async_copy(x_hbm.at[pl.ds(off, _CS)],
                                 rows_v.at[bsel], sem_ld),
                pltpu.async_copy(sb_hbm.at[pl.ds(off, _CS)],
                                 srow_v.at[bsel], sem_ld),
            )

        ld = loads(0, 0)
        st_prev = None
        for j in range(_NCH):
            bj = j % 2
            for h in ld:
                h.wait()
            if st_prev is not None:
                for h in st_prev:
                    h.wait()
            if j + 1 < _NCH:
                ld = loads(j + 1, (j + 1) % 2)
            st_prev = (
                pltpu.async_copy(rows_v.at[bj], out_hbm.at[idx_v.at[bj]],
                                 sem_st),
                pltpu.async_copy(srow_v.at[bj], ss_hbm.at[idx_v.at[bj]],
                                 sem_st),
            )
        for h in st_prev:
            h.wait()

    return k(x, score_b, slot)


def _sc_gather(out_sorted, slot):
    mesh = plsc.VectorSubcoreMesh(core_axis_name="c", subcore_axis_name="s")

    @functools.partial(
        pl.kernel,
        out_type=jax.ShapeDtypeStruct((T, D), _f32),
        mesh=mesh,
        scratch_types=[
            pltpu.VMEM((2, _CS), _i32),
            pltpu.VMEM((2, _CS, D), _f32),
            pltpu.SemaphoreType.DMA,
            pltpu.SemaphoreType.DMA,
            pltpu.SemaphoreType.DMA,
        ],
    )
    def k(src_hbm, slot_hbm, out_hbm, idx_v, rows_v, sem_ld, sem_g, sem_st):
        wid = lax.axis_index("s") * _NC + lax.axis_index("c")
        base = wid * _PW

        def idx_load(j, bsel):
            off = base + j * _CS
            return pltpu.async_copy(slot_hbm.at[pl.ds(off, _CS)],
                                    idx_v.at[bsel], sem_ld)

        ld = idx_load(0, 0)
        g_prev = None
        st_prev = None
        for j in range(_NCH):
            bj = j % 2
            ld.wait()
            if st_prev is not None:
                st_prev.wait()
            if j + 1 < _NCH:
                ld = idx_load(j + 1, (j + 1) % 2)
            g = pltpu.async_copy(src_hbm.at[idx_v.at[bj]], rows_v.at[bj],
                                 sem_g)
            g.wait()
            off = base + j * _CS
            st_prev = pltpu.async_copy(rows_v.at[bj],
                                       out_hbm.at[pl.ds(off, _CS)], sem_st)
        st_prev.wait()

    return k(out_sorted, slot)


def _upr_body(be_ref, x_ref, ss_ref, wg_ref, wu_ref, h_ref, wg16, wu16):
    b = pl.program_id(1)
    e = be_ref[b]
    eprev = be_ref[jnp.maximum(b - 1, 0)]

    @pl.when(e != BE_NONE)
    def _():
        @pl.when((b == 0) | (e != eprev))
        def _():
            wg16[...] = wg_ref[0].astype(_bf16)
            wu16[...] = wu_ref[0].astype(_bf16)

        x16 = (x_ref[...] * ss_ref[:, :1]).astype(_bf16)
        g = jnp.dot(x16, wg16[...], preferred_element_type=_f32)
        u = jnp.dot(x16, wu16[...], preferred_element_type=_f32)
        h = g / (1.0 + jnp.exp(-g)) * u
        h_ref[...] = h.astype(_bf16)


def _up_routed(be, x_sorted, ssort, w_gate, w_up):
    grid_spec = pltpu.PrefetchScalarGridSpec(
        num_scalar_prefetch=1,
        grid=(NF, NB),
        in_specs=[
            pl.BlockSpec((BQ, D),
                         lambda fo, b, be: (jnp.where(be[b] == BE_NONE, 0, b),
                                            0)),
            pl.BlockSpec((BQ, EP),
                         lambda fo, b, be: (jnp.where(be[b] == BE_NONE, 0, b),
                                            0)),
            pl.BlockSpec((1, D, TF),
                         lambda fo, b, be: (jnp.minimum(be[b], E - 1), 0, fo)),
            pl.BlockSpec((1, D, TF),
                         lambda fo, b, be: (jnp.minimum(be[b], E - 1), 0, fo)),
        ],
        out_specs=pl.BlockSpec((BQ, TF), lambda fo, b, be: (b, fo)),
        scratch_shapes=[pltpu.VMEM((D, TF), _bf16),
                        pltpu.VMEM((D, TF), _bf16)],
    )
    return pl.pallas_call(
        _upr_body,
        grid_spec=grid_spec,
        out_shape=jax.ShapeDtypeStruct((NBQ, F), _bf16),
        compiler_params=pltpu.CompilerParams(
            dimension_semantics=("arbitrary", "arbitrary")),
    )(be, x_sorted, ssort, w_gate, w_up)


def _downr_body(be_ref, h_ref, wd_ref, o_ref, wd16):
    b = pl.program_id(1)
    e = be_ref[b]
    eprev = be_ref[jnp.maximum(b - 1, 0)]

    @pl.when(e != BE_NONE)
    def _():
        @pl.when((b == 0) | (e != eprev))
        def _():
            wd16[...] = wd_ref[0].astype(_bf16)

        o_ref[...] = jnp.dot(h_ref[...], wd16[...],
                             preferred_element_type=_f32)


def _down_routed(be, h, w_down):
    grid_spec = pltpu.PrefetchScalarGridSpec(
        num_scalar_prefetch=1,
        grid=(ND, NB),
        in_specs=[
            pl.BlockSpec((BQ, F),
                         lambda dc, b, be: (jnp.where(be[b] == BE_NONE, 0, b),
                                            0)),
            pl.BlockSpec((1, F, TD),
                         lambda dc, b, be: (jnp.minimum(be[b], E - 1), 0, dc)),
        ],
        out_specs=pl.BlockSpec((BQ, TD), lambda dc, b, be: (b, dc)),
        scratch_shapes=[pltpu.VMEM((F, TD), _bf16)],
    )
    return pl.pallas_call(
        _downr_body,
        grid_spec=grid_spec,
        out_shape=jax.ShapeDtypeStruct((NBQ, D), _f32),
        compiler_params=pltpu.CompilerParams(
            dimension_semantics=("arbitrary", "arbitrary")),
    )(be, h, w_down)


def _ups_body(x_ref, wg_ref, wu_ref, h_ref, wg16, wu16):
    tb = pl.program_id(1)

    @pl.when(tb == 0)
    def _():
        wg16[...] = wg_ref[...].astype(_bf16)
        wu16[...] = wu_ref[...].astype(_bf16)

    x16 = x_ref[...].astype(_bf16)
    g = jnp.dot(x16, wg16[...], preferred_element_type=_f32)
    u = jnp.dot(x16, wu16[...], preferred_element_type=_f32)
    h = g / (1.0 + jnp.exp(-g)) * u
    h_ref[...] = h.astype(_bf16)


def _up_shared(x, ws_gate, ws_up):
    return pl.pallas_call(
        _ups_body,
        grid=(NF, NTB),
        in_specs=[
            pl.BlockSpec((BQ, D), lambda fo, tb: (tb, 0)),
            pl.BlockSpec((D, TF), lambda fo, tb: (0, fo)),
            pl.BlockSpec((D, TF), lambda fo, tb: (0, fo)),
        ],
        out_specs=pl.BlockSpec((BQ, TF), lambda fo, tb: (tb, fo)),
        out_shape=jax.ShapeDtypeStruct((T, F), _bf16),
        scratch_shapes=[pltpu.VMEM((D, TF), _bf16),
                        pltpu.VMEM((D, TF), _bf16)],
        compiler_params=pltpu.CompilerParams(
            dimension_semantics=("arbitrary", "arbitrary")),
    )(x, ws_gate, ws_up)


def _downs_body(h_ref, wd_ref, rg_ref, o_ref, wd16):
    tb = pl.program_id(1)

    @pl.when(tb == 0)
    def _():
        wd16[...] = wd_ref[...].astype(_bf16)

    o_ref[...] = (jnp.dot(h_ref[...], wd16[...], preferred_element_type=_f32)
                  + rg_ref[...])


def _down_shared(h, ws_down, routed):
    return pl.pallas_call(
        _downs_body,
        grid=(ND, NTB),
        in_specs=[
            pl.BlockSpec((BQ, F), lambda dc, tb: (tb, 0)),
            pl.BlockSpec((F, TD), lambda dc, tb: (0, dc)),
            pl.BlockSpec((BQ, TD), lambda dc, tb: (tb, dc)),
        ],
        out_specs=pl.BlockSpec((BQ, TD), lambda dc, tb: (tb, dc)),
        out_shape=jax.ShapeDtypeStruct((T, D), _f32),
        scratch_shapes=[pltpu.VMEM((F, TD), _bf16)],
        compiler_params=pltpu.CompilerParams(
            dimension_semantics=("arbitrary", "arbitrary")),
    )(h, ws_down, routed)


def kernel(hidden_states, W_router, W_gate, W_up, W_down,
           Ws_gate, Ws_up, Ws_down):
    wr_pad = jnp.pad(W_router, ((0, 0), (0, EP - E)))
    score_b, slot2, be = _router(hidden_states, wr_pad)
    slot = slot2.reshape(T)
    be_flat = be[:NB, 0]

    x_sorted, ssort = _sc_scatter(hidden_states, score_b, slot)
    h = _up_routed(be_flat, x_sorted, ssort, W_gate, W_up)
    out_sorted = _down_routed(be_flat, h, W_down)
    routed = _sc_gather(out_sorted, slot)

    hs = _up_shared(hidden_states, Ws_gate, Ws_up)
    return _down_shared(hs, Ws_down, routed)
